# baseline (speedup 1.0000x reference)
import functools

import jax
import jax.numpy as jnp
from jax import lax
from jax.experimental import pallas as pl
from jax.experimental.pallas import tpu as pltpu
from jax.experimental.pallas import tpu_sc as plsc

_PLACEHOLDER = 42
_B, _N, _D = 4, 8192, 1024
_R = _B * _N

_info = plsc.get_sparse_core_info()
_NC, _NS = _info.num_cores, _info.num_subcores
_NW = _NC * _NS
_RPW = _R // _NW
_CH = 32
_NBUF = 3
_ITERS = _RPW // _CH


def _matvec_body(s_ref, w_ref, b_ref, o_ref):
    o_ref[...] = (
        jnp.dot(s_ref[...], w_ref[...], preferred_element_type=jnp.float32)
        + b_ref[...]
    )


_P = 8


@functools.partial(
    pl.kernel,
    out_type=jax.ShapeDtypeStruct((_R, _D), jnp.float32),
    mesh=plsc.VectorSubcoreMesh(core_axis_name="c", subcore_axis_name="s"),
    compiler_params=pltpu.CompilerParams(needs_layout_passes=False),
    scratch_types=[
        pltpu.VMEM((_RPW,), jnp.int32),
        pltpu.VMEM((1, _D), jnp.float32),
    ]
    + [pltpu.SemaphoreType.DMA] * _P,
)
def _sc_body(tok_hbm, x_hbm, rep_hbm, out_hbm, tok_v, rep_v, *sems):
    wid = lax.axis_index("s") * _NC + lax.axis_index("c")
    base = wid * _RPW

    pltpu.sync_copy(tok_hbm.at[pl.ds(base, _RPW)], tok_v)
    pltpu.sync_copy(rep_hbm, rep_v)

    rows = _RPW // _P
    copies = [
        pltpu.make_async_copy(
            x_hbm.at[pl.ds(base + p * rows, rows)],
            out_hbm.at[pl.ds(base + p * rows, rows)],
            sems[p],
        )
        for p in range(_P)
    ]
    for c in copies:
        c.start()
    for c in copies:
        c.wait()

    iota = lax.iota(jnp.int32, 16)
    zeros = jnp.zeros((16,), jnp.int32)
    ones = jnp.full((16,), 1, jnp.int32)
    ph = jnp.full((16,), _PLACEHOLDER, jnp.int32)

    def _group(g, carry):
        tv = tok_v[pl.ds(g * 16, 16)]
        m = jnp.where(tv == ph, ones, zeros)
        cnt = jnp.sum(m)

        @pl.when(cnt > 0)
        def _scan_lanes():
            def _lane(l, c2):
                lb = lax.broadcast_in_dim(l, (16,), ())
                hit = jnp.sum(jnp.where(iota == lb, m, zeros))

                @pl.when(hit > 0)
                def _dma():
                    pltpu.sync_copy(
                        rep_v, out_hbm.at[pl.ds(base + g * 16 + l, 1)])

                return c2

            lax.fori_loop(0, 16, _lane, 0)

        return carry

    lax.fori_loop(0, _RPW // 16, _group, 0)


def kernel(tokenized_text, embedded_text, special_embeddings, W, b):
    B, N, D = embedded_text.shape
    R = B * N
    x = embedded_text.reshape(R, D)
    tok = tokenized_text.reshape(R).astype(jnp.int32)
    s = special_embeddings.reshape(1, D)
    bias = b.reshape(1, D)

    rep = pl.pallas_call(
        _matvec_body,
        out_shape=jax.ShapeDtypeStruct((1, D), jnp.float32),
    )(s, W, bias)

    out = _sc_body(tok, x, rep)
    return out.reshape(B, N, D)


# speedup vs baseline: 34.8917x; 34.8917x over previous
import functools

import jax
import jax.numpy as jnp
from jax import lax
from jax.experimental import pallas as pl
from jax.experimental.pallas import tpu as pltpu
from jax.experimental.pallas import tpu_sc as plsc

_PLACEHOLDER = 42
_B, _N, _D = 4, 8192, 1024
_R = _B * _N

_info = plsc.get_sparse_core_info()
_NC, _NS = _info.num_cores, _info.num_subcores
_NW = _NC * _NS
_RPW = _R // _NW
_CH = 16
_NBUF = 6
_L = 3
_ITERS = _RPW // _CH


def _matvec_body(s_ref, w_ref, b_ref, o_ref):
    o_ref[...] = (
        jnp.dot(s_ref[...], w_ref[...], preferred_element_type=jnp.float32)
        + b_ref[...]
    )


@functools.partial(
    pl.kernel,
    out_type=jax.ShapeDtypeStruct((_R, _D), jnp.float32),
    mesh=plsc.VectorSubcoreMesh(core_axis_name="c", subcore_axis_name="s"),
    compiler_params=pltpu.CompilerParams(needs_layout_passes=False),
    scratch_types=[
        pltpu.VMEM((_RPW,), jnp.int32),
        pltpu.VMEM((1, _D), jnp.float32),
        pltpu.VMEM((_NBUF, _CH, _D), jnp.float32),
    ]
    + [pltpu.SemaphoreType.DMA] * (2 * _NBUF),
)
def _sc_body(tok_hbm, x_hbm, rep_hbm, out_hbm, tok_v, rep_v, buf, *sems):
    in_sems, st_sems = sems[:_NBUF], sems[_NBUF:]
    wid = lax.axis_index("s") * _NC + lax.axis_index("c")
    base = wid * _RPW

    pltpu.sync_copy(tok_hbm.at[pl.ds(base, _RPW)], tok_v)
    pltpu.sync_copy(rep_hbm, rep_v)

    def in_copy(it):
        k = it % _NBUF
        return pltpu.make_async_copy(
            x_hbm.at[pl.ds(base + it * _CH, _CH)], buf.at[k], in_sems[k])

    def st_copy(it):
        k = it % _NBUF
        return pltpu.make_async_copy(
            buf.at[k], out_hbm.at[pl.ds(base + it * _CH, _CH)], st_sems[k])

    for k in range(_L):
        in_copy(k).start()
    for it in range(_ITERS):
        in_copy(it).wait()
        st_copy(it).start()
        nxt = it + _L
        if nxt < _ITERS:
            if nxt - _NBUF >= 0:
                st_copy(nxt - _NBUF).wait()
            in_copy(nxt).start()
    for it in range(_ITERS - _NBUF, _ITERS):
        st_copy(it).wait()

    zeros = jnp.zeros((16,), jnp.int32)
    ones = jnp.full((16,), 1, jnp.int32)
    ph = jnp.full((16,), _PLACEHOLDER, jnp.int32)

    iota = lax.iota(jnp.int32, 16)
    zeros = jnp.zeros((16,), jnp.int32)
    ones = jnp.full((16,), 1, jnp.int32)
    ph = jnp.full((16,), _PLACEHOLDER, jnp.int32)

    def _group(g, carry):
        tv = tok_v[pl.ds(g * 16, 16)]
        m = jnp.where(tv == ph, ones, zeros)
        cnt = jnp.sum(m)

        @pl.when(cnt > 0)
        def _scan_lanes():
            def _lane(l, c2):
                lb = lax.broadcast_in_dim(l, (16,), ())
                hit = jnp.sum(jnp.where(iota == lb, m, zeros))

                @pl.when(hit > 0)
                def _dma():
                    pltpu.sync_copy(
                        rep_v, out_hbm.at[pl.ds(base + g * 16 + l, 1)])

                return c2

            lax.fori_loop(0, 16, _lane, 0)

        return carry

    lax.fori_loop(0, _RPW // 16, _group, 0)


def kernel(tokenized_text, embedded_text, special_embeddings, W, b):
    B, N, D = embedded_text.shape
    R = B * N
    x = embedded_text.reshape(R, D)
    tok = tokenized_text.reshape(R).astype(jnp.int32)
    s = special_embeddings.reshape(1, D)
    bias = b.reshape(1, D)

    rep = pl.pallas_call(
        _matvec_body,
        out_shape=jax.ShapeDtypeStruct((1, D), jnp.float32),
    )(s, W, bias)

    out = _sc_body(tok, x, rep)
    return out.reshape(B, N, D)


# speedup vs baseline: 48.3325x; 1.3852x over previous
import jax
import jax.numpy as jnp
from jax.experimental import pallas as pl
from jax.experimental.pallas import tpu as pltpu

_PLACEHOLDER = 42
_BN = 2048
_C = 128


def _body(tok_ref, x_ref, s_ref, w_ref, b_ref, o_ref, rep_ref, eye_ref):
    i = pl.program_id(0)

    @pl.when(i == 0)
    def _init():
        rep_ref[...] = (
            jnp.dot(s_ref[...], w_ref[...], preferred_element_type=jnp.float32)
            + b_ref[...]
        )
        rows = jax.lax.broadcasted_iota(jnp.int32, (_C, _C), 0)
        cols = jax.lax.broadcasted_iota(jnp.int32, (_C, _C), 1)
        eye_ref[...] = (rows == cols).astype(jnp.float32)

    N = tok_ref.shape[1]
    r = i * _BN
    rep = rep_ref[...]
    tok_row = tok_ref[pl.ds(r // N, 1), pl.ds(r % N, _BN)]
    for c in range(_BN // _C):
        tok = jax.lax.slice(tok_row, (0, c * _C), (1, (c + 1) * _C))
        maskf = (tok == _PLACEHOLDER).astype(jnp.float32)
        mask_col = jax.lax.dot_general(
            eye_ref[...], maskf,
            dimension_numbers=(((1,), (1,)), ((), ())),
            preferred_element_type=jnp.float32,
        )
        sl = pl.ds(c * _C, _C)
        o_ref[sl, :] = jnp.where(mask_col > 0.5, rep, x_ref[sl, :])


def kernel(tokenized_text, embedded_text, special_embeddings, W, b):
    B, N, D = embedded_text.shape
    R = B * N
    x = embedded_text.reshape(R, D)
    tok = tokenized_text.astype(jnp.int32)
    s = special_embeddings.reshape(1, D)
    bias = b.reshape(1, D)

    out = pl.pallas_call(
        _body,
        grid=(R // _BN,),
        in_specs=[
            pl.BlockSpec((B, N), lambda i: (0, 0)),
            pl.BlockSpec((_BN, D), lambda i: (i, 0)),
            pl.BlockSpec((1, D), lambda i: (0, 0)),
            pl.BlockSpec((D, D), lambda i: (0, 0)),
            pl.BlockSpec((1, D), lambda i: (0, 0)),
        ],
        out_specs=pl.BlockSpec((_BN, D), lambda i: (i, 0)),
        out_shape=jax.ShapeDtypeStruct((R, D), jnp.float32),
        scratch_shapes=[
            pltpu.VMEM((1, D), jnp.float32),
            pltpu.VMEM((_C, _C), jnp.float32),
        ],
    )(tok, x, s, W, bias)
    return out.reshape(B, N, D)
